# baseline (speedup 1.0000x reference)
import functools

import jax
import jax.numpy as jnp
from jax import lax
from jax.experimental import pallas as pl
from jax.experimental.pallas import tpu as pltpu

B, P, K = 8, 2048, 20
TP = 512
BIG = 1e30
HI = jax.lax.Precision.HIGHEST


def _hilo(x):
    hi = x.astype(jnp.bfloat16)
    lo = (x - hi.astype(jnp.float32)).astype(jnp.bfloat16)
    return hi, lo


def _gather_mm(onehot_bf, hi_ref, lo_ref):
    g = jnp.dot(onehot_bf, hi_ref[...], preferred_element_type=jnp.float32)
    return g + jnp.dot(onehot_bf, lo_ref[...],
                       preferred_element_type=jnp.float32)


def _conv1_body(posr_ref, post_ref, m1_ref, m2_ref, c1_ref, w1c_ref, b1c_ref,
                x1_ref, qh_ref, ql_ref):
    t = pl.program_id(1)

    @pl.when(t == 0)
    def _():
        q = jnp.dot(posr_ref[0], m2_ref[...], precision=HI)
        qh, ql = _hilo(q)
        qh_ref[...] = qh
        ql_ref[...] = ql

    post = post_ref[0]
    pos_tile = posr_ref[0, pl.ds(t * TP, TP), :]
    s_lane = jnp.sum(post * post, axis=0, keepdims=True)
    d2 = s_lane - 2.0 * jnp.dot(pos_tile, post, precision=HI)

    lane = lax.broadcasted_iota(jnp.int32, (TP, P), 1)
    row = lax.broadcasted_iota(jnp.int32, (TP, P), 0) + t * TP
    d2 = jnp.where(lane == row, BIG, d2)

    p_tile = jnp.dot(pos_tile, m1_ref[...], precision=HI) + c1_ref[...]
    w1c = w1c_ref[...]
    b1c = b1c_ref[...]

    def body(_, carry):
        d2c, acc = carry
        m = jnp.min(d2c, axis=1, keepdims=True)
        tie = jnp.where(d2c == m, lane, P)
        amin = jnp.min(tie, axis=1, keepdims=True)
        onehot = (lane == amin).astype(jnp.bfloat16)
        qn = _gather_mm(onehot, qh_ref, ql_ref)
        tt = jnp.maximum(p_tile + qn, 0.0)
        s = jnp.maximum(jnp.dot(tt, w1c, precision=HI) + b1c, 0.0)
        return jnp.where(lane == amin, BIG, d2c), jnp.maximum(acc, s)

    acc0 = jnp.full((TP, 64), -BIG, jnp.float32)
    carry = (d2, acc0)
    for _ in range(K):
        carry = body(None, carry)
    _, acc = carry
    x1_ref[0] = acc


def _conv2_body(x1_ref, x1t_ref, u2_ref, b2_ref, v2_ref, x2_ref,
                vh_ref, vl_ref):
    t = pl.program_id(1)

    @pl.when(t == 0)
    def _():
        v2 = jnp.dot(x1_ref[0], v2_ref[...], precision=HI)
        vh, vl = _hilo(v2)
        vh_ref[...] = vh
        vl_ref[...] = vl

    x1t = x1t_ref[0]
    x1_tile = x1_ref[0, pl.ds(t * TP, TP), :]
    s_lane = jnp.sum(x1t * x1t, axis=0, keepdims=True)
    d2 = s_lane - 2.0 * jnp.dot(x1_tile, x1t, precision=HI)

    lane = lax.broadcasted_iota(jnp.int32, (TP, P), 1)
    row = lax.broadcasted_iota(jnp.int32, (TP, P), 0) + t * TP
    d2 = jnp.where(lane == row, BIG, d2)

    u2_tile = jnp.dot(x1_tile, u2_ref[...], precision=HI) + b2_ref[...]

    def body(_, carry):
        d2c, acc = carry
        m = jnp.min(d2c, axis=1, keepdims=True)
        tie = jnp.where(d2c == m, lane, P)
        amin = jnp.min(tie, axis=1, keepdims=True)
        onehot = (lane == amin).astype(jnp.bfloat16)
        vn = _gather_mm(onehot, vh_ref, vl_ref)
        return jnp.where(lane == amin, BIG, d2c), jnp.maximum(acc, vn)

    acc0 = jnp.full((TP, 128), -BIG, jnp.float32)
    carry = (d2, acc0)
    for _ in range(K):
        carry = body(None, carry)
    _, acc = carry
    x2_ref[0] = jnp.maximum(u2_tile + acc, 0.0)


def _head_body(x1_ref, x2_ref, wl1a_ref, wl1b_ref, bl1_ref,
               wm1_ref, bm1_ref, wm2_ref, bm2_ref, wm3_ref, bm3_ref,
               out_ref):
    a = jnp.dot(x1_ref[0], wl1a_ref[...], precision=HI)
    a = a + jnp.dot(x2_ref[0], wl1b_ref[...], precision=HI)
    a = jnp.maximum(a + bl1_ref[...], 0.0)
    g = jnp.max(a, axis=0, keepdims=True)
    h = jnp.maximum(jnp.dot(g, wm1_ref[...], precision=HI) + bm1_ref[...], 0.0)
    h = jnp.maximum(jnp.dot(h, wm2_ref[...], precision=HI) + bm2_ref[...], 0.0)
    o = jnp.dot(h, wm3_ref[...], precision=HI) + bm3_ref[...]
    o = o - jnp.max(o, axis=1, keepdims=True)
    out_ref[0] = o - jnp.log(jnp.sum(jnp.exp(o), axis=1, keepdims=True))


def kernel(pos, batch, W1a, b1a, W1b, b1b, W1c, b1c, W2, b2, Wl1, bl1,
           Wm1, bm1, Wm2, bm2, Wm3, bm3):
    del batch
    f32 = jnp.float32
    x = pos.reshape(B, P, 3).astype(f32)
    posr = jnp.pad(x, ((0, 0), (0, 0), (0, 5)))
    post = jnp.transpose(posr, (0, 2, 1))

    A, Bm = W1a[:3], W1a[3:]
    M1 = jnp.pad((A - Bm) @ W1b, ((0, 5), (0, 0)))
    M2 = jnp.pad(Bm @ W1b, ((0, 5), (0, 0)))
    c1 = (b1a @ W1b + b1b).reshape(1, 64)
    b1cr = b1c.reshape(1, 64)

    x1 = pl.pallas_call(
        _conv1_body,
        grid=(B, P // TP),
        in_specs=[
            pl.BlockSpec((1, P, 8), lambda b, t: (b, 0, 0)),
            pl.BlockSpec((1, 8, P), lambda b, t: (b, 0, 0)),
            pl.BlockSpec((8, 64), lambda b, t: (0, 0)),
            pl.BlockSpec((8, 64), lambda b, t: (0, 0)),
            pl.BlockSpec((1, 64), lambda b, t: (0, 0)),
            pl.BlockSpec((64, 64), lambda b, t: (0, 0)),
            pl.BlockSpec((1, 64), lambda b, t: (0, 0)),
        ],
        out_specs=pl.BlockSpec((1, TP, 64), lambda b, t: (b, t, 0)),
        out_shape=jax.ShapeDtypeStruct((B, P, 64), f32),
        scratch_shapes=[pltpu.VMEM((P, 64), jnp.bfloat16),
                        pltpu.VMEM((P, 64), jnp.bfloat16)],
    )(posr, post, M1, M2, c1, W1c, b1cr)

    x1t = jnp.transpose(x1, (0, 2, 1))
    U2 = W2[:64] - W2[64:]
    V2 = W2[64:]
    b2r = b2.reshape(1, 128)

    x2 = pl.pallas_call(
        _conv2_body,
        grid=(B, P // TP),
        in_specs=[
            pl.BlockSpec((1, P, 64), lambda b, t: (b, 0, 0)),
            pl.BlockSpec((1, 64, P), lambda b, t: (b, 0, 0)),
            pl.BlockSpec((64, 128), lambda b, t: (0, 0)),
            pl.BlockSpec((1, 128), lambda b, t: (0, 0)),
            pl.BlockSpec((64, 128), lambda b, t: (0, 0)),
        ],
        out_specs=pl.BlockSpec((1, TP, 128), lambda b, t: (b, t, 0)),
        out_shape=jax.ShapeDtypeStruct((B, P, 128), f32),
        scratch_shapes=[pltpu.VMEM((P, 128), jnp.bfloat16),
                        pltpu.VMEM((P, 128), jnp.bfloat16)],
    )(x1, x1t, U2, b2r, V2)

    out = pl.pallas_call(
        _head_body,
        grid=(B,),
        in_specs=[
            pl.BlockSpec((1, P, 64), lambda b: (b, 0, 0)),
            pl.BlockSpec((1, P, 128), lambda b: (b, 0, 0)),
            pl.BlockSpec((64, 1024), lambda b: (0, 0)),
            pl.BlockSpec((128, 1024), lambda b: (0, 0)),
            pl.BlockSpec((1, 1024), lambda b: (0, 0)),
            pl.BlockSpec((1024, 512), lambda b: (0, 0)),
            pl.BlockSpec((1, 512), lambda b: (0, 0)),
            pl.BlockSpec((512, 256), lambda b: (0, 0)),
            pl.BlockSpec((1, 256), lambda b: (0, 0)),
            pl.BlockSpec((256, 40), lambda b: (0, 0)),
            pl.BlockSpec((1, 40), lambda b: (0, 0)),
        ],
        out_specs=pl.BlockSpec((1, 1, 40), lambda b: (b, 0, 0)),
        out_shape=jax.ShapeDtypeStruct((B, 1, 40), f32),
    )(x1, x2, Wl1[:64], Wl1[64:], bl1.reshape(1, 1024),
      Wm1, bm1.reshape(1, 512), Wm2, bm2.reshape(1, 256),
      Wm3, bm3.reshape(1, 40))

    return out.reshape(B, 40)


# speedup vs baseline: 1.1122x; 1.1122x over previous
import functools

import jax
import jax.numpy as jnp
from jax import lax
from jax.experimental import pallas as pl
from jax.experimental.pallas import tpu as pltpu

B, P, K = 8, 2048, 20
TP = 128
BIG = 1e30
HI = jax.lax.Precision.HIGHEST


def _hilo(x):
    hi = x.astype(jnp.bfloat16)
    lo = (x - hi.astype(jnp.float32)).astype(jnp.bfloat16)
    return hi, lo


def _gather_mm(onehot_bf, hi_ref, lo_ref):
    g = jnp.dot(onehot_bf, hi_ref[...], preferred_element_type=jnp.float32)
    return g + jnp.dot(onehot_bf, lo_ref[...],
                       preferred_element_type=jnp.float32)


def _conv1_body(posr_ref, post_ref, m1_ref, m2_ref, c1_ref, w1c_ref, b1c_ref,
                x1_ref, qh_ref, ql_ref):
    t = pl.program_id(1)

    @pl.when(t == 0)
    def _():
        q = jnp.dot(posr_ref[0], m2_ref[...], precision=HI)
        qh, ql = _hilo(q)
        qh_ref[...] = qh
        ql_ref[...] = ql

    post = post_ref[0]
    pos_tile = posr_ref[0, pl.ds(t * TP, TP), :]
    s_lane = jnp.sum(post * post, axis=0, keepdims=True)
    d2 = s_lane - 2.0 * jnp.dot(pos_tile, post, precision=HI)

    lane = lax.broadcasted_iota(jnp.int32, (TP, P), 1)
    row = lax.broadcasted_iota(jnp.int32, (TP, P), 0) + t * TP
    d2 = jnp.where(lane == row, BIG, d2)

    p_tile = jnp.dot(pos_tile, m1_ref[...], precision=HI) + c1_ref[...]
    w1c = w1c_ref[...]
    b1c = b1c_ref[...]

    def body(_, carry):
        d2c, acc = carry
        m = jnp.min(d2c, axis=1, keepdims=True)
        tie = jnp.where(d2c == m, lane, P)
        amin = jnp.min(tie, axis=1, keepdims=True)
        onehot = (lane == amin).astype(jnp.bfloat16)
        qn = _gather_mm(onehot, qh_ref, ql_ref)
        tt = jnp.maximum(p_tile + qn, 0.0)
        s = jnp.maximum(jnp.dot(tt, w1c, precision=HI) + b1c, 0.0)
        return jnp.where(lane == amin, BIG, d2c), jnp.maximum(acc, s)

    acc0 = jnp.full((TP, 64), -BIG, jnp.float32)
    carry = (d2, acc0)
    for _ in range(K):
        carry = body(None, carry)
    _, acc = carry
    x1_ref[0] = acc


def _conv2_body(x1_ref, x1t_ref, u2_ref, b2_ref, v2_ref, x2_ref,
                vh_ref, vl_ref):
    t = pl.program_id(1)

    @pl.when(t == 0)
    def _():
        v2 = jnp.dot(x1_ref[0], v2_ref[...], precision=HI)
        vh, vl = _hilo(v2)
        vh_ref[...] = vh
        vl_ref[...] = vl

    x1t = x1t_ref[0]
    x1_tile = x1_ref[0, pl.ds(t * TP, TP), :]
    s_lane = jnp.sum(x1t * x1t, axis=0, keepdims=True)
    d2 = s_lane - 2.0 * jnp.dot(x1_tile, x1t, precision=HI)

    lane = lax.broadcasted_iota(jnp.int32, (TP, P), 1)
    row = lax.broadcasted_iota(jnp.int32, (TP, P), 0) + t * TP
    d2 = jnp.where(lane == row, BIG, d2)

    u2_tile = jnp.dot(x1_tile, u2_ref[...], precision=HI) + b2_ref[...]

    def body(_, carry):
        d2c, acc = carry
        m = jnp.min(d2c, axis=1, keepdims=True)
        tie = jnp.where(d2c == m, lane, P)
        amin = jnp.min(tie, axis=1, keepdims=True)
        onehot = (lane == amin).astype(jnp.bfloat16)
        vn = _gather_mm(onehot, vh_ref, vl_ref)
        return jnp.where(lane == amin, BIG, d2c), jnp.maximum(acc, vn)

    acc0 = jnp.full((TP, 128), -BIG, jnp.float32)
    carry = (d2, acc0)
    for _ in range(K):
        carry = body(None, carry)
    _, acc = carry
    x2_ref[0] = jnp.maximum(u2_tile + acc, 0.0)


def _head_body(x1_ref, x2_ref, wl1a_ref, wl1b_ref, bl1_ref,
               wm1_ref, bm1_ref, wm2_ref, bm2_ref, wm3_ref, bm3_ref,
               out_ref):
    a = jnp.dot(x1_ref[0], wl1a_ref[...], precision=HI)
    a = a + jnp.dot(x2_ref[0], wl1b_ref[...], precision=HI)
    a = jnp.maximum(a + bl1_ref[...], 0.0)
    g = jnp.max(a, axis=0, keepdims=True)
    h = jnp.maximum(jnp.dot(g, wm1_ref[...], precision=HI) + bm1_ref[...], 0.0)
    h = jnp.maximum(jnp.dot(h, wm2_ref[...], precision=HI) + bm2_ref[...], 0.0)
    o = jnp.dot(h, wm3_ref[...], precision=HI) + bm3_ref[...]
    o = o - jnp.max(o, axis=1, keepdims=True)
    out_ref[0] = o - jnp.log(jnp.sum(jnp.exp(o), axis=1, keepdims=True))


def kernel(pos, batch, W1a, b1a, W1b, b1b, W1c, b1c, W2, b2, Wl1, bl1,
           Wm1, bm1, Wm2, bm2, Wm3, bm3):
    del batch
    f32 = jnp.float32
    x = pos.reshape(B, P, 3).astype(f32)
    posr = jnp.pad(x, ((0, 0), (0, 0), (0, 5)))
    post = jnp.transpose(posr, (0, 2, 1))

    A, Bm = W1a[:3], W1a[3:]
    M1 = jnp.pad((A - Bm) @ W1b, ((0, 5), (0, 0)))
    M2 = jnp.pad(Bm @ W1b, ((0, 5), (0, 0)))
    c1 = (b1a @ W1b + b1b).reshape(1, 64)
    b1cr = b1c.reshape(1, 64)

    x1 = pl.pallas_call(
        _conv1_body,
        grid=(B, P // TP),
        in_specs=[
            pl.BlockSpec((1, P, 8), lambda b, t: (b, 0, 0)),
            pl.BlockSpec((1, 8, P), lambda b, t: (b, 0, 0)),
            pl.BlockSpec((8, 64), lambda b, t: (0, 0)),
            pl.BlockSpec((8, 64), lambda b, t: (0, 0)),
            pl.BlockSpec((1, 64), lambda b, t: (0, 0)),
            pl.BlockSpec((64, 64), lambda b, t: (0, 0)),
            pl.BlockSpec((1, 64), lambda b, t: (0, 0)),
        ],
        out_specs=pl.BlockSpec((1, TP, 64), lambda b, t: (b, t, 0)),
        out_shape=jax.ShapeDtypeStruct((B, P, 64), f32),
        scratch_shapes=[pltpu.VMEM((P, 64), jnp.bfloat16),
                        pltpu.VMEM((P, 64), jnp.bfloat16)],
    )(posr, post, M1, M2, c1, W1c, b1cr)

    x1t = jnp.transpose(x1, (0, 2, 1))
    U2 = W2[:64] - W2[64:]
    V2 = W2[64:]
    b2r = b2.reshape(1, 128)

    x2 = pl.pallas_call(
        _conv2_body,
        grid=(B, P // TP),
        in_specs=[
            pl.BlockSpec((1, P, 64), lambda b, t: (b, 0, 0)),
            pl.BlockSpec((1, 64, P), lambda b, t: (b, 0, 0)),
            pl.BlockSpec((64, 128), lambda b, t: (0, 0)),
            pl.BlockSpec((1, 128), lambda b, t: (0, 0)),
            pl.BlockSpec((64, 128), lambda b, t: (0, 0)),
        ],
        out_specs=pl.BlockSpec((1, TP, 128), lambda b, t: (b, t, 0)),
        out_shape=jax.ShapeDtypeStruct((B, P, 128), f32),
        scratch_shapes=[pltpu.VMEM((P, 128), jnp.bfloat16),
                        pltpu.VMEM((P, 128), jnp.bfloat16)],
    )(x1, x1t, U2, b2r, V2)

    out = pl.pallas_call(
        _head_body,
        grid=(B,),
        in_specs=[
            pl.BlockSpec((1, P, 64), lambda b: (b, 0, 0)),
            pl.BlockSpec((1, P, 128), lambda b: (b, 0, 0)),
            pl.BlockSpec((64, 1024), lambda b: (0, 0)),
            pl.BlockSpec((128, 1024), lambda b: (0, 0)),
            pl.BlockSpec((1, 1024), lambda b: (0, 0)),
            pl.BlockSpec((1024, 512), lambda b: (0, 0)),
            pl.BlockSpec((1, 512), lambda b: (0, 0)),
            pl.BlockSpec((512, 256), lambda b: (0, 0)),
            pl.BlockSpec((1, 256), lambda b: (0, 0)),
            pl.BlockSpec((256, 40), lambda b: (0, 0)),
            pl.BlockSpec((1, 40), lambda b: (0, 0)),
        ],
        out_specs=pl.BlockSpec((1, 1, 40), lambda b: (b, 0, 0)),
        out_shape=jax.ShapeDtypeStruct((B, 1, 40), f32),
    )(x1, x2, Wl1[:64], Wl1[64:], bl1.reshape(1, 1024),
      Wm1, bm1.reshape(1, 512), Wm2, bm2.reshape(1, 256),
      Wm3, bm3.reshape(1, 40))

    return out.reshape(B, 40)


# speedup vs baseline: 1.3485x; 1.2124x over previous
import functools

import jax
import jax.numpy as jnp
from jax import lax
from jax.experimental import pallas as pl
from jax.experimental.pallas import tpu as pltpu

B, P, K = 8, 2048, 20
TP = 256
BIG = 1e30
HI = jax.lax.Precision.HIGHEST


def _hilo(x):
    hi = x.astype(jnp.bfloat16)
    lo = (x - hi.astype(jnp.float32)).astype(jnp.bfloat16)
    return hi, lo


def _gather_mm(onehot_bf, hi_ref, lo_ref):
    g = jnp.dot(onehot_bf, hi_ref[...], preferred_element_type=jnp.float32)
    return g + jnp.dot(onehot_bf, lo_ref[...],
                       preferred_element_type=jnp.float32)


MAXI = 0x7FFFFFFF


def _packed_keys(x_tile, xt_all, t):
    s_i = jnp.sum(x_tile * x_tile, axis=1, keepdims=True)
    s_j = jnp.sum(xt_all * xt_all, axis=0, keepdims=True)
    d2 = s_i + s_j - 2.0 * jnp.dot(x_tile, xt_all, precision=HI)
    d2 = jnp.maximum(d2, 0.0)
    bits = lax.bitcast_convert_type(d2, jnp.int32)
    lane = lax.broadcasted_iota(jnp.int32, d2.shape, 1)
    row = lax.broadcasted_iota(jnp.int32, d2.shape, 0) + t * TP
    ckey = jnp.bitwise_or(jnp.bitwise_and(bits, -2048), lane)
    return jnp.where(lane == row, MAXI, ckey)


def _conv1_body(posr_ref, post_ref, m1_ref, m2_ref, c1_ref, w1c_ref, b1c_ref,
                x1_ref, qh_ref, ql_ref):
    t = pl.program_id(1)

    @pl.when(t == 0)
    def _():
        q = jnp.dot(posr_ref[0], m2_ref[...], precision=HI)
        qh, ql = _hilo(q)
        qh_ref[...] = qh
        ql_ref[...] = ql

    post = post_ref[0]
    pos_tile = posr_ref[0, pl.ds(t * TP, TP), :]
    ckey = _packed_keys(pos_tile, post, t)

    p_tile = jnp.dot(pos_tile, m1_ref[...], precision=HI) + c1_ref[...]
    w1c = w1c_ref[...]
    b1c = b1c_ref[...]

    def body(_, carry):
        ck, acc = carry
        mk = jnp.min(ck, axis=1, keepdims=True)
        eq = ck == mk
        onehot = eq.astype(jnp.bfloat16)
        qn = _gather_mm(onehot, qh_ref, ql_ref)
        tt = jnp.maximum(p_tile + qn, 0.0)
        s = jnp.maximum(jnp.dot(tt, w1c, precision=HI) + b1c, 0.0)
        return jnp.where(eq, MAXI, ck), jnp.maximum(acc, s)

    acc0 = jnp.full((TP, 64), -BIG, jnp.float32)
    carry = (ckey, acc0)
    for _ in range(K):
        carry = body(None, carry)
    _, acc = carry
    x1_ref[0] = acc


def _conv2_body(x1_ref, x1t_ref, u2_ref, b2_ref, v2_ref, x2_ref,
                vh_ref, vl_ref):
    t = pl.program_id(1)

    @pl.when(t == 0)
    def _():
        v2 = jnp.dot(x1_ref[0], v2_ref[...], precision=HI)
        vh, vl = _hilo(v2)
        vh_ref[...] = vh
        vl_ref[...] = vl

    x1t = x1t_ref[0]
    x1_tile = x1_ref[0, pl.ds(t * TP, TP), :]
    ckey = _packed_keys(x1_tile, x1t, t)

    u2_tile = jnp.dot(x1_tile, u2_ref[...], precision=HI) + b2_ref[...]

    def body(_, carry):
        ck, acc = carry
        mk = jnp.min(ck, axis=1, keepdims=True)
        eq = ck == mk
        onehot = eq.astype(jnp.bfloat16)
        vn = _gather_mm(onehot, vh_ref, vl_ref)
        return jnp.where(eq, MAXI, ck), jnp.maximum(acc, vn)

    acc0 = jnp.full((TP, 128), -BIG, jnp.float32)
    carry = (ckey, acc0)
    for _ in range(K):
        carry = body(None, carry)
    _, acc = carry
    x2_ref[0] = jnp.maximum(u2_tile + acc, 0.0)


def _head_body(x1_ref, x2_ref, wl1a_ref, wl1b_ref, bl1_ref,
               wm1_ref, bm1_ref, wm2_ref, bm2_ref, wm3_ref, bm3_ref,
               out_ref):
    a = jnp.dot(x1_ref[0], wl1a_ref[...], precision=HI)
    a = a + jnp.dot(x2_ref[0], wl1b_ref[...], precision=HI)
    a = jnp.maximum(a + bl1_ref[...], 0.0)
    g = jnp.max(a, axis=0, keepdims=True)
    h = jnp.maximum(jnp.dot(g, wm1_ref[...], precision=HI) + bm1_ref[...], 0.0)
    h = jnp.maximum(jnp.dot(h, wm2_ref[...], precision=HI) + bm2_ref[...], 0.0)
    o = jnp.dot(h, wm3_ref[...], precision=HI) + bm3_ref[...]
    o = o - jnp.max(o, axis=1, keepdims=True)
    out_ref[0] = o - jnp.log(jnp.sum(jnp.exp(o), axis=1, keepdims=True))


def kernel(pos, batch, W1a, b1a, W1b, b1b, W1c, b1c, W2, b2, Wl1, bl1,
           Wm1, bm1, Wm2, bm2, Wm3, bm3):
    del batch
    f32 = jnp.float32
    x = pos.reshape(B, P, 3).astype(f32)
    posr = jnp.pad(x, ((0, 0), (0, 0), (0, 5)))
    post = jnp.transpose(posr, (0, 2, 1))

    A, Bm = W1a[:3], W1a[3:]
    M1 = jnp.pad((A - Bm) @ W1b, ((0, 5), (0, 0)))
    M2 = jnp.pad(Bm @ W1b, ((0, 5), (0, 0)))
    c1 = (b1a @ W1b + b1b).reshape(1, 64)
    b1cr = b1c.reshape(1, 64)

    x1 = pl.pallas_call(
        _conv1_body,
        grid=(B, P // TP),
        in_specs=[
            pl.BlockSpec((1, P, 8), lambda b, t: (b, 0, 0)),
            pl.BlockSpec((1, 8, P), lambda b, t: (b, 0, 0)),
            pl.BlockSpec((8, 64), lambda b, t: (0, 0)),
            pl.BlockSpec((8, 64), lambda b, t: (0, 0)),
            pl.BlockSpec((1, 64), lambda b, t: (0, 0)),
            pl.BlockSpec((64, 64), lambda b, t: (0, 0)),
            pl.BlockSpec((1, 64), lambda b, t: (0, 0)),
        ],
        out_specs=pl.BlockSpec((1, TP, 64), lambda b, t: (b, t, 0)),
        out_shape=jax.ShapeDtypeStruct((B, P, 64), f32),
        scratch_shapes=[pltpu.VMEM((P, 64), jnp.bfloat16),
                        pltpu.VMEM((P, 64), jnp.bfloat16)],
    )(posr, post, M1, M2, c1, W1c, b1cr)

    x1t = jnp.transpose(x1, (0, 2, 1))
    U2 = W2[:64] - W2[64:]
    V2 = W2[64:]
    b2r = b2.reshape(1, 128)

    x2 = pl.pallas_call(
        _conv2_body,
        grid=(B, P // TP),
        in_specs=[
            pl.BlockSpec((1, P, 64), lambda b, t: (b, 0, 0)),
            pl.BlockSpec((1, 64, P), lambda b, t: (b, 0, 0)),
            pl.BlockSpec((64, 128), lambda b, t: (0, 0)),
            pl.BlockSpec((1, 128), lambda b, t: (0, 0)),
            pl.BlockSpec((64, 128), lambda b, t: (0, 0)),
        ],
        out_specs=pl.BlockSpec((1, TP, 128), lambda b, t: (b, t, 0)),
        out_shape=jax.ShapeDtypeStruct((B, P, 128), f32),
        scratch_shapes=[pltpu.VMEM((P, 128), jnp.bfloat16),
                        pltpu.VMEM((P, 128), jnp.bfloat16)],
    )(x1, x1t, U2, b2r, V2)

    out = pl.pallas_call(
        _head_body,
        grid=(B,),
        in_specs=[
            pl.BlockSpec((1, P, 64), lambda b: (b, 0, 0)),
            pl.BlockSpec((1, P, 128), lambda b: (b, 0, 0)),
            pl.BlockSpec((64, 1024), lambda b: (0, 0)),
            pl.BlockSpec((128, 1024), lambda b: (0, 0)),
            pl.BlockSpec((1, 1024), lambda b: (0, 0)),
            pl.BlockSpec((1024, 512), lambda b: (0, 0)),
            pl.BlockSpec((1, 512), lambda b: (0, 0)),
            pl.BlockSpec((512, 256), lambda b: (0, 0)),
            pl.BlockSpec((1, 256), lambda b: (0, 0)),
            pl.BlockSpec((256, 40), lambda b: (0, 0)),
            pl.BlockSpec((1, 40), lambda b: (0, 0)),
        ],
        out_specs=pl.BlockSpec((1, 1, 40), lambda b: (b, 0, 0)),
        out_shape=jax.ShapeDtypeStruct((B, 1, 40), f32),
    )(x1, x2, Wl1[:64], Wl1[64:], bl1.reshape(1, 1024),
      Wm1, bm1.reshape(1, 512), Wm2, bm2.reshape(1, 256),
      Wm3, bm3.reshape(1, 40))

    return out.reshape(B, 40)


# speedup vs baseline: 1.8852x; 1.3980x over previous
import functools

import jax
import jax.numpy as jnp
from jax import lax
from jax.experimental import pallas as pl
from jax.experimental.pallas import tpu as pltpu

B, P, K = 8, 2048, 20
TP = 256
BIG = 1e30
HI = jax.lax.Precision.HIGHEST


def _hilo(x):
    hi = x.astype(jnp.bfloat16)
    lo = (x - hi.astype(jnp.float32)).astype(jnp.bfloat16)
    return hi, lo


def _gather_mm(onehot_bf, cat_ref, n):
    g = jnp.dot(onehot_bf, cat_ref[...], preferred_element_type=jnp.float32)
    return g[:, :n] + g[:, n:]


MAXI = 0x7FFFFFFF


def _packed_keys(x_tile, xt_all, t):
    s_i = jnp.sum(x_tile * x_tile, axis=1, keepdims=True)
    s_j = jnp.sum(xt_all * xt_all, axis=0, keepdims=True)
    d2 = s_i + s_j - 2.0 * jnp.dot(x_tile, xt_all, precision=HI)
    d2 = jnp.maximum(d2, 0.0)
    bits = lax.bitcast_convert_type(d2, jnp.int32)
    lane = lax.broadcasted_iota(jnp.int32, d2.shape, 1)
    row = lax.broadcasted_iota(jnp.int32, d2.shape, 0) + t * TP
    ckey = jnp.bitwise_or(jnp.bitwise_and(bits, -2048), lane)
    return jnp.where(lane == row, MAXI, ckey)


def _conv1_body(posr_ref, post_ref, m1_ref, m2_ref, c1_ref, w1c_ref, b1c_ref,
                x1_ref, qc_ref):
    t = pl.program_id(1)

    @pl.when(t == 0)
    def _():
        q = jnp.dot(posr_ref[0], m2_ref[...], precision=HI)
        qh, ql = _hilo(q)
        qc_ref[...] = jnp.concatenate([qh, ql], axis=1)

    post = post_ref[0]
    pos_tile = posr_ref[0, pl.ds(t * TP, TP), :]
    ckey = _packed_keys(pos_tile, post, t)

    p_tile = jnp.dot(pos_tile, m1_ref[...], precision=HI) + c1_ref[...]
    w1c = w1c_ref[...]
    b1c = b1c_ref[...]

    def body(_, carry):
        ck, acc = carry
        mk = jnp.min(ck, axis=1, keepdims=True)
        eq = ck == mk
        onehot = eq.astype(jnp.bfloat16)
        qn = _gather_mm(onehot, qc_ref, 64)
        tt = jnp.maximum(p_tile + qn, 0.0)
        s = jnp.maximum(jnp.dot(tt, w1c, precision=HI) + b1c, 0.0)
        return jnp.where(eq, MAXI, ck), jnp.maximum(acc, s)

    acc0 = jnp.full((TP, 64), -BIG, jnp.float32)
    carry = (ckey, acc0)
    for _ in range(K):
        carry = body(None, carry)
    _, acc = carry
    x1_ref[0] = acc


def _conv2_body(x1_ref, x1t_ref, u2_ref, b2_ref, v2_ref, x2_ref,
                vc_ref):
    t = pl.program_id(1)

    @pl.when(t == 0)
    def _():
        v2 = jnp.dot(x1_ref[0], v2_ref[...], precision=HI)
        vh, vl = _hilo(v2)
        vc_ref[...] = jnp.concatenate([vh, vl], axis=1)

    x1t = x1t_ref[0]
    x1_tile = x1_ref[0, pl.ds(t * TP, TP), :]
    ckey = _packed_keys(x1_tile, x1t, t)

    u2_tile = jnp.dot(x1_tile, u2_ref[...], precision=HI) + b2_ref[...]

    def body(_, carry):
        ck, acc = carry
        mk = jnp.min(ck, axis=1, keepdims=True)
        eq = ck == mk
        onehot = eq.astype(jnp.bfloat16)
        vn = _gather_mm(onehot, vc_ref, 128)
        return jnp.where(eq, MAXI, ck), jnp.maximum(acc, vn)

    acc0 = jnp.full((TP, 128), -BIG, jnp.float32)
    carry = (ckey, acc0)
    for _ in range(K):
        carry = body(None, carry)
    _, acc = carry
    x2_ref[0] = jnp.maximum(u2_tile + acc, 0.0)


def _head_body(x1_ref, x2_ref, wl1a_ref, wl1b_ref, bl1_ref,
               wm1_ref, bm1_ref, wm2_ref, bm2_ref, wm3_ref, bm3_ref,
               out_ref):
    a = jnp.dot(x1_ref[0], wl1a_ref[...], precision=HI)
    a = a + jnp.dot(x2_ref[0], wl1b_ref[...], precision=HI)
    a = jnp.maximum(a + bl1_ref[...], 0.0)
    g = jnp.max(a, axis=0, keepdims=True)
    h = jnp.maximum(jnp.dot(g, wm1_ref[...], precision=HI) + bm1_ref[...], 0.0)
    h = jnp.maximum(jnp.dot(h, wm2_ref[...], precision=HI) + bm2_ref[...], 0.0)
    o = jnp.dot(h, wm3_ref[...], precision=HI) + bm3_ref[...]
    o = o - jnp.max(o, axis=1, keepdims=True)
    out_ref[0] = o - jnp.log(jnp.sum(jnp.exp(o), axis=1, keepdims=True))


def kernel(pos, batch, W1a, b1a, W1b, b1b, W1c, b1c, W2, b2, Wl1, bl1,
           Wm1, bm1, Wm2, bm2, Wm3, bm3):
    del batch
    f32 = jnp.float32
    x = pos.reshape(B, P, 3).astype(f32)
    posr = jnp.pad(x, ((0, 0), (0, 0), (0, 5)))
    post = jnp.transpose(posr, (0, 2, 1))

    A, Bm = W1a[:3], W1a[3:]
    M1 = jnp.pad((A - Bm) @ W1b, ((0, 5), (0, 0)))
    M2 = jnp.pad(Bm @ W1b, ((0, 5), (0, 0)))
    c1 = (b1a @ W1b + b1b).reshape(1, 64)
    b1cr = b1c.reshape(1, 64)

    x1 = pl.pallas_call(
        _conv1_body,
        grid=(B, P // TP),
        in_specs=[
            pl.BlockSpec((1, P, 8), lambda b, t: (b, 0, 0)),
            pl.BlockSpec((1, 8, P), lambda b, t: (b, 0, 0)),
            pl.BlockSpec((8, 64), lambda b, t: (0, 0)),
            pl.BlockSpec((8, 64), lambda b, t: (0, 0)),
            pl.BlockSpec((1, 64), lambda b, t: (0, 0)),
            pl.BlockSpec((64, 64), lambda b, t: (0, 0)),
            pl.BlockSpec((1, 64), lambda b, t: (0, 0)),
        ],
        out_specs=pl.BlockSpec((1, TP, 64), lambda b, t: (b, t, 0)),
        out_shape=jax.ShapeDtypeStruct((B, P, 64), f32),
        scratch_shapes=[pltpu.VMEM((P, 128), jnp.bfloat16)],
    )(posr, post, M1, M2, c1, W1c, b1cr)

    x1t = jnp.transpose(x1, (0, 2, 1))
    U2 = W2[:64] - W2[64:]
    V2 = W2[64:]
    b2r = b2.reshape(1, 128)

    x2 = pl.pallas_call(
        _conv2_body,
        grid=(B, P // TP),
        in_specs=[
            pl.BlockSpec((1, P, 64), lambda b, t: (b, 0, 0)),
            pl.BlockSpec((1, 64, P), lambda b, t: (b, 0, 0)),
            pl.BlockSpec((64, 128), lambda b, t: (0, 0)),
            pl.BlockSpec((1, 128), lambda b, t: (0, 0)),
            pl.BlockSpec((64, 128), lambda b, t: (0, 0)),
        ],
        out_specs=pl.BlockSpec((1, TP, 128), lambda b, t: (b, t, 0)),
        out_shape=jax.ShapeDtypeStruct((B, P, 128), f32),
        scratch_shapes=[pltpu.VMEM((P, 256), jnp.bfloat16)],
    )(x1, x1t, U2, b2r, V2)

    out = pl.pallas_call(
        _head_body,
        grid=(B,),
        in_specs=[
            pl.BlockSpec((1, P, 64), lambda b: (b, 0, 0)),
            pl.BlockSpec((1, P, 128), lambda b: (b, 0, 0)),
            pl.BlockSpec((64, 1024), lambda b: (0, 0)),
            pl.BlockSpec((128, 1024), lambda b: (0, 0)),
            pl.BlockSpec((1, 1024), lambda b: (0, 0)),
            pl.BlockSpec((1024, 512), lambda b: (0, 0)),
            pl.BlockSpec((1, 512), lambda b: (0, 0)),
            pl.BlockSpec((512, 256), lambda b: (0, 0)),
            pl.BlockSpec((1, 256), lambda b: (0, 0)),
            pl.BlockSpec((256, 40), lambda b: (0, 0)),
            pl.BlockSpec((1, 40), lambda b: (0, 0)),
        ],
        out_specs=pl.BlockSpec((1, 1, 40), lambda b: (b, 0, 0)),
        out_shape=jax.ShapeDtypeStruct((B, 1, 40), f32),
    )(x1, x2, Wl1[:64], Wl1[64:], bl1.reshape(1, 1024),
      Wm1, bm1.reshape(1, 512), Wm2, bm2.reshape(1, 256),
      Wm3, bm3.reshape(1, 40))

    return out.reshape(B, 40)
